# Initial kernel scaffold; baseline (speedup 1.0000x reference)
import jax
import jax.numpy as jnp
from jax.experimental import pallas as pl


def kernel(lstm_scores, word_seq_lens, tags, mask, transition):
    raise NotImplementedError("write your pallas kernel here")


# speedup vs baseline: 11.0711x; 11.0711x over previous
import functools

import jax
import jax.numpy as jnp
from jax.experimental import pallas as pl

B, L, K = 16, 512, 64
START_IDX, END_IDX, PAD_IDX = 61, 62, 63


def _crf_body(scores_t_ref, tags_t_ref, prev_t_ref, maskf_t_ref,
              wsl_col_ref, wsl_row_ref, transition_ref,
              out_u_ref, out_l_ref):
    trans = transition_ref[:, :]
    exp_t = jnp.exp(trans)

    alpha0 = trans[START_IDX:START_IDX + 1, :] + scores_t_ref[0]
    wsl_col = wsl_col_ref[:, :]

    def step(t, carry):
        alpha, last_alpha = carry
        m = jnp.max(alpha, axis=1, keepdims=True)
        e = jnp.exp(alpha - m)
        s = jnp.dot(e, exp_t, preferred_element_type=jnp.float32)
        alpha_new = jnp.log(s) + m + scores_t_ref[t]
        is_last = (wsl_col - 1) == t
        last_alpha = jnp.where(is_last, alpha_new, last_alpha)
        return alpha_new, last_alpha

    alpha, last_alpha = jax.lax.fori_loop(1, L, step, (alpha0, alpha0))

    m2 = jnp.max(last_alpha, axis=1, keepdims=True)
    e2 = jnp.exp(last_alpha - m2)
    v = jnp.dot(e2, exp_t, preferred_element_type=jnp.float32)
    oh_end = (jax.lax.broadcasted_iota(jnp.int32, (B, K), 1) == END_IDX)
    picked = jnp.sum(jnp.where(oh_end, v, 0.0), axis=1, keepdims=True)
    ub = jnp.log(picked) + m2
    out_u_ref[:, :] = jnp.sum(ub, axis=0, keepdims=True)

    tags_t = tags_t_ref[:, :]
    prev_t = prev_t_ref[:, :]
    maskf = maskf_t_ref[:, :]

    iota_k3 = jax.lax.broadcasted_iota(jnp.int32, (L, B, K), 2)
    oh_tag = (tags_t[:, :, None] == iota_k3).astype(jnp.float32)
    oh_prev = (prev_t[:, :, None] == iota_k3).astype(jnp.float32)

    oh_prev2 = jnp.reshape(oh_prev, (L * B, K))
    u = jnp.dot(oh_prev2, trans, preferred_element_type=jnp.float32)
    u3 = jnp.reshape(u, (L, B, K))
    per_elem = jnp.sum(oh_tag * (u3 + scores_t_ref[:, :, :]), axis=2)
    seq_sum = jnp.sum(per_elem * maskf)

    iota_l = jax.lax.broadcasted_iota(jnp.int32, (L, B), 0)
    is_last_t = (iota_l == (wsl_row_ref[:, :] - 1)).astype(jnp.int32)
    last_tag = jnp.sum(tags_t * is_last_t, axis=0, keepdims=True)
    iota_kb = jax.lax.broadcasted_iota(jnp.int32, (K, B), 0)
    oh_last = (last_tag == iota_kb).astype(jnp.float32)
    cnt = jnp.sum(oh_last, axis=1, keepdims=True)
    end_sum = jnp.sum(cnt * trans[:, END_IDX:END_IDX + 1])

    total = seq_sum + end_sum
    out_l_ref[:, :] = jnp.reshape(total, (1, 1))


@jax.jit
def kernel(lstm_scores, word_seq_lens, tags, mask, transition):
    scores_t = jnp.transpose(lstm_scores, (1, 0, 2))
    tags_t = jnp.transpose(tags, (1, 0))
    prev = jnp.concatenate(
        [jnp.full((B, 1), START_IDX, dtype=tags.dtype), tags[:, :-1]], axis=1)
    prev_t = jnp.transpose(prev, (1, 0))
    maskf_t = jnp.transpose(mask.astype(jnp.float32), (1, 0))
    wsl_col = word_seq_lens.reshape(B, 1)
    wsl_row = word_seq_lens.reshape(1, B)

    out_u, out_l = pl.pallas_call(
        _crf_body,
        out_shape=[
            jax.ShapeDtypeStruct((1, 1), jnp.float32),
            jax.ShapeDtypeStruct((1, 1), jnp.float32),
        ],
    )(scores_t, tags_t, prev_t, maskf_t, wsl_col, wsl_row, transition)
    return (out_u.reshape(()), out_l.reshape(()))


# speedup vs baseline: 17.4705x; 1.5780x over previous
import functools

import jax
import jax.numpy as jnp
from jax.experimental import pallas as pl

B, L, K = 16, 512, 64
START_IDX, END_IDX, PAD_IDX = 61, 62, 63


def _crf_body(scores_t_ref, tags_t_ref, prev_t_ref, maskf_t_ref,
              wsl_col_ref, wsl_row_ref, transition_ref,
              out_u_ref, out_l_ref):
    trans = transition_ref[:, :]
    max_t = jnp.max(trans)
    exp_ts = jnp.exp(trans - max_t)

    wsl_col = wsl_col_ref[:, :]

    a0_log = trans[START_IDX:START_IDX + 1, :] + scores_t_ref[0]
    m0 = jnp.max(a0_log, axis=1, keepdims=True)
    a = jnp.exp(a0_log - m0)
    off = m0
    m_prev = jnp.max(a, axis=1, keepdims=True)

    def one_step(t, state):
        a, m_prev, off, last_a, last_off = state
        s = jnp.dot(a, exp_ts, preferred_element_type=jnp.float32)
        g = jnp.exp(scores_t_ref[t]) * (1.0 / m_prev)
        a_new = s * g
        off_new = off + (jnp.log(m_prev) + max_t)
        is_last = (wsl_col - 1) == t
        last_a = jnp.where(is_last, a_new, last_a)
        last_off = jnp.where(is_last, off_new, last_off)
        m_new = jnp.max(a_new, axis=1, keepdims=True)
        return a_new, m_new, off_new, last_a, last_off

    state = (a, m_prev, off, a, off)
    state = one_step(1, state)
    state = one_step(2, state)
    state = one_step(3, state)

    def body4(i, state):
        for j in range(4):
            state = one_step(4 * i + j, state)
        return state

    _, _, _, last_a, last_off = jax.lax.fori_loop(1, L // 4, body4, state)

    v = jnp.dot(last_a, exp_ts, preferred_element_type=jnp.float32)
    oh_end = (jax.lax.broadcasted_iota(jnp.int32, (B, K), 1) == END_IDX)
    picked = jnp.sum(jnp.where(oh_end, v, 0.0), axis=1, keepdims=True)
    ub = jnp.log(picked) + last_off + max_t
    out_u_ref[:, :] = jnp.sum(ub, axis=0, keepdims=True)

    tags_t = tags_t_ref[:, :]
    prev_t = prev_t_ref[:, :]
    maskf = maskf_t_ref[:, :]

    iota_k3 = jax.lax.broadcasted_iota(jnp.int32, (L, B, K), 2)
    oh_tag = (tags_t[:, :, None] == iota_k3).astype(jnp.float32)
    oh_prev = (prev_t[:, :, None] == iota_k3).astype(jnp.float32)

    oh_prev2 = jnp.reshape(oh_prev, (L * B, K))
    u = jnp.dot(oh_prev2, trans, preferred_element_type=jnp.float32)
    u3 = jnp.reshape(u, (L, B, K))
    per_elem = jnp.sum(oh_tag * (u3 + scores_t_ref[:, :, :]), axis=2)
    seq_sum = jnp.sum(per_elem * maskf)

    iota_l = jax.lax.broadcasted_iota(jnp.int32, (L, B), 0)
    is_last_t = (iota_l == (wsl_row_ref[:, :] - 1)).astype(jnp.int32)
    last_tag = jnp.sum(tags_t * is_last_t, axis=0, keepdims=True)
    iota_kb = jax.lax.broadcasted_iota(jnp.int32, (K, B), 0)
    oh_last = (last_tag == iota_kb).astype(jnp.float32)
    cnt = jnp.sum(oh_last, axis=1, keepdims=True)
    end_sum = jnp.sum(cnt * trans[:, END_IDX:END_IDX + 1])

    total = seq_sum + end_sum
    out_l_ref[:, :] = jnp.reshape(total, (1, 1))


@jax.jit
def kernel(lstm_scores, word_seq_lens, tags, mask, transition):
    scores_t = jnp.transpose(lstm_scores, (1, 0, 2))
    tags_t = jnp.transpose(tags, (1, 0))
    prev = jnp.concatenate(
        [jnp.full((B, 1), START_IDX, dtype=tags.dtype), tags[:, :-1]], axis=1)
    prev_t = jnp.transpose(prev, (1, 0))
    maskf_t = jnp.transpose(mask.astype(jnp.float32), (1, 0))
    wsl_col = word_seq_lens.reshape(B, 1)
    wsl_row = word_seq_lens.reshape(1, B)

    out_u, out_l = pl.pallas_call(
        _crf_body,
        out_shape=[
            jax.ShapeDtypeStruct((1, 1), jnp.float32),
            jax.ShapeDtypeStruct((1, 1), jnp.float32),
        ],
    )(scores_t, tags_t, prev_t, maskf_t, wsl_col, wsl_row, transition)
    return (out_u.reshape(()), out_l.reshape(()))
